# baseline (speedup 1.0000x reference)
import functools

import jax
import jax.numpy as jnp
from jax.experimental import pallas as pl
from jax.experimental.pallas import tpu as pltpu

_MARGIN = 0.2
_EPS = 1e-12


def _normalize(x, eps):
    n = jnp.sqrt(jnp.sum(x * x, axis=1, keepdims=True))
    return x / jnp.maximum(n, eps)


def _fold_min(v, bm, cj):
    parts = [v[:, k * 128:(k + 1) * 128] for k in range(cj // 128)]
    while len(parts) > 1:
        nxt = [jnp.minimum(parts[t], parts[t + 1])
               for t in range(0, len(parts) - 1, 2)]
        if len(parts) % 2:
            nxt.append(parts[-1])
        parts = nxt
    return parts[0]


def _consume(vref, jblk, i, min_ref, bm, cj):
    v = vref[...]
    jd = (i * bm) // cj

    @pl.when(jblk == jd)
    def _masked():
        row_g = i * bm + jax.lax.broadcasted_iota(jnp.int32, (bm, cj), 0)
        col_g = jblk * cj + jax.lax.broadcasted_iota(jnp.int32, (bm, cj), 1)
        vm = jnp.where(row_g == col_g, jnp.inf, v)
        min_ref[...] = jnp.minimum(min_ref[...], _fold_min(vm, bm, cj))

    @pl.when(jblk != jd)
    def _plain():
        min_ref[...] = jnp.minimum(min_ref[...], _fold_min(v, bm, cj))


def _triplet_body(a_ref, p_ref, pd_ref, out_ref,
                  paug_ref, aaug_ref, asq_ref, ap_ref, min_ref,
                  bufa_ref, bufb_ref,
                  *, bm, cj, n, nj, d, margin, eps):
    i = pl.program_id(0)
    j = pl.program_id(1)

    @pl.when((i == 0) & (j == 0))
    def _build_paug():
        pn = _normalize(p_ref[...], eps)
        paug_ref[:, :d] = -2.0 * pn
        paug_ref[:, d:] = jnp.sum(pn * pn, axis=1, keepdims=True)

    @pl.when(j == 0)
    def _build_aaug():
        an_ = _normalize(a_ref[...], eps)
        aaug_ref[:, :d] = an_
        aaug_ref[:, d:] = jnp.ones((bm, 1), jnp.float32)
        asq_ref[...] = jnp.sum(an_ * an_, axis=1, keepdims=True)
        pdn = _normalize(pd_ref[...], eps)
        ap_ref[...] = jnp.sum((an_ - pdn) * (an_ - pdn), axis=1,
                              keepdims=True)
        min_ref[...] = jnp.full((bm, 128), jnp.inf, jnp.float32)

    mm = jax.lax.dot_general(
        aaug_ref[...], paug_ref[pl.ds(j * cj, cj), :],
        (((1,), (1,)), ((), ())),
        preferred_element_type=jnp.float32)

    slot = jax.lax.rem(j, 2)

    @pl.when(slot == 0)
    def _even():
        bufa_ref[...] = mm

        @pl.when(j > 0)
        def _():
            _consume(bufb_ref, j - 1, i, min_ref, bm, cj)

    @pl.when(slot == 1)
    def _odd():
        bufb_ref[...] = mm
        _consume(bufa_ref, j - 1, i, min_ref, bm, cj)

    @pl.when(j == nj - 1)
    def _finalize():
        tail_ref = bufb_ref if (nj - 1) % 2 else bufa_ref
        _consume(tail_ref, j, i, min_ref, bm, cj)
        rowmin = jnp.min(min_ref[...], axis=1, keepdims=True)
        an_dist = asq_ref[...] + rowmin
        losses = jnp.maximum(ap_ref[...] - an_dist + margin, 0.0)
        part = jnp.sum(losses, keepdims=True) * (1.0 / n)

        @pl.when(i == 0)
        def _init_out():
            out_ref[...] = jnp.zeros_like(out_ref)

        out_ref[...] += part


@jax.jit
def kernel(anchors, positives):
    n, d = anchors.shape
    bm = 256
    cj = 2048
    ni, nj = n // bm, n // cj
    body = functools.partial(_triplet_body, bm=bm, cj=cj, n=n, nj=nj, d=d,
                             margin=_MARGIN, eps=_EPS)
    out = pl.pallas_call(
        body,
        grid=(ni, nj),
        in_specs=[
            pl.BlockSpec((bm, d), lambda i, j: (i, 0)),
            pl.BlockSpec((n, d), lambda i, j: (0, 0)),
            pl.BlockSpec((bm, d), lambda i, j: (i, 0)),
        ],
        out_specs=pl.BlockSpec((1, 1), lambda i, j: (0, 0)),
        out_shape=jax.ShapeDtypeStruct((1, 1), jnp.float32),
        scratch_shapes=[
            pltpu.VMEM((n, d + 1), jnp.float32),
            pltpu.VMEM((bm, d + 1), jnp.float32),
            pltpu.VMEM((bm, 1), jnp.float32),
            pltpu.VMEM((bm, 1), jnp.float32),
            pltpu.VMEM((bm, 128), jnp.float32),
            pltpu.VMEM((bm, cj), jnp.float32),
            pltpu.VMEM((bm, cj), jnp.float32),
        ],
    )(anchors, positives, positives)
    return out[0, 0]


# speedup vs baseline: 2.5511x; 2.5511x over previous
import functools

import jax
import jax.numpy as jnp
from jax.experimental import pallas as pl
from jax.experimental.pallas import tpu as pltpu

_MARGIN = 0.2
_EPS = 1e-12


def _normalize(x, eps):
    n = jnp.sqrt(jnp.sum(x * x, axis=1, keepdims=True))
    return x / jnp.maximum(n, eps)


def _min_tree(parts):
    parts = list(parts)
    while len(parts) > 1:
        nxt = [jnp.minimum(parts[t], parts[t + 1])
               for t in range(0, len(parts) - 1, 2)]
        if len(parts) % 2:
            nxt.append(parts[-1])
        parts = nxt
    return parts[0]


def _fold_min(v, cj):
    return _min_tree([v[:, k * 128:(k + 1) * 128] for k in range(cj // 128)])


def _triplet_body(a_ref, p_ref, pd_ref, out_ref,
                  paug_ref, aaug_ref, mask_ref,
                  *, bm, cj, n, nj, d, margin, eps):
    i = pl.program_id(0)

    @pl.when(i == 0)
    def _setup():
        pn = _normalize(p_ref[...], eps)
        paug_ref[:, :d] = -2.0 * pn
        paug_ref[:, d:] = jnp.sum(pn * pn, axis=1, keepdims=True)
        row = jax.lax.broadcasted_iota(jnp.int32, (bm, 2 * cj), 0)
        col = jax.lax.broadcasted_iota(jnp.int32, (bm, 2 * cj), 1)
        mask_ref[...] = jnp.where(col == row + cj, jnp.inf, 0.0)
        out_ref[...] = jnp.zeros_like(out_ref)

    a_n = _normalize(a_ref[...], eps)
    aaug_ref[:, :d] = a_n
    aaug_ref[:, d:] = jnp.ones((bm, 1), jnp.float32)
    a_sq = jnp.sum(a_n * a_n, axis=1, keepdims=True)
    pdn = _normalize(pd_ref[...], eps)
    ap = jnp.sum((a_n - pdn) * (a_n - pdn), axis=1, keepdims=True)
    aaug = aaug_ref[...]

    jd = (i * bm) // cj
    off = i * bm - jd * cj

    def tile(u):
        blk = jax.lax.rem(jd + u, nj)
        mm = jax.lax.dot_general(
            aaug, paug_ref[pl.ds(blk * cj, cj), :],
            (((1,), (1,)), ((), ())),
            preferred_element_type=jnp.float32)
        if u == 0:
            mm = mm + mask_ref[pl.ds(0, bm), pl.ds(cj - off, cj)]
        return _fold_min(mm, cj)

    folded = _min_tree([tile(u) for u in range(nj)])
    rowmin = jnp.min(folded, axis=1, keepdims=True)
    losses = jnp.maximum(ap - (a_sq + rowmin) + margin, 0.0)
    out_ref[...] += jnp.sum(losses, keepdims=True) * (1.0 / n)


@jax.jit
def kernel(anchors, positives):
    n, d = anchors.shape
    bm = 256
    cj = 2048
    ni, nj = n // bm, n // cj
    body = functools.partial(_triplet_body, bm=bm, cj=cj, n=n, nj=nj, d=d,
                             margin=_MARGIN, eps=_EPS)
    out = pl.pallas_call(
        body,
        grid=(ni,),
        in_specs=[
            pl.BlockSpec((bm, d), lambda i: (i, 0)),
            pl.BlockSpec((n, d), lambda i: (0, 0)),
            pl.BlockSpec((bm, d), lambda i: (i, 0)),
        ],
        out_specs=pl.BlockSpec((1, 1), lambda i: (0, 0)),
        out_shape=jax.ShapeDtypeStruct((1, 1), jnp.float32),
        scratch_shapes=[
            pltpu.VMEM((n, d + 1), jnp.float32),
            pltpu.VMEM((bm, d + 1), jnp.float32),
            pltpu.VMEM((bm, 2 * cj), jnp.float32),
        ],
    )(anchors, positives, positives)
    return out[0, 0]
